# baseline (speedup 1.0000x reference)
import functools
import math

import jax
import jax.numpy as jnp
import numpy as np
from jax import lax
from jax.experimental import pallas as pl
from jax.experimental.pallas import tpu as pltpu
from jax.experimental.pallas import tpu_sc as plsc

VOCAB = 1000000
D_MODEL = 64
B = 4096
L = 200
N = B * L

NUM_CORES = 2
NUM_SUBCORES = 16
NUM_WORKERS = NUM_CORES * NUM_SUBCORES
BATCH_BLOCK = B // NUM_WORKERS
LANES = 16
PADDED_D = 128
JGROUPS = D_MODEL // LANES


def _make_pe(max_len, d_model):
    pos = np.arange(max_len, dtype=np.float32)[:, None]
    div = np.exp(
        np.arange(0, d_model, 2, dtype=np.float32) * (-math.log(10000.0) / d_model)
    )
    pe = np.zeros((max_len, d_model), dtype=np.float32)
    pe[:, 0::2] = np.sin(pos * div)
    pe[:, 1::2] = np.cos(pos * div)
    return pe


_PE = _make_pe(L, D_MODEL)


@functools.partial(
    pl.kernel,
    mesh=plsc.VectorSubcoreMesh(core_axis_name="c", subcore_axis_name="s"),
    compiler_params=pltpu.CompilerParams(
        use_tc_tiling_on_sc=True, needs_layout_passes=False),
    out_type=jax.ShapeDtypeStruct((L, D_MODEL, B), jnp.float32),
    scratch_types=[
        pltpu.VMEM((L * D_MODEL,), jnp.float32),
        [pltpu.VMEM((BATCH_BLOCK,), jnp.int32)] * 2,
        [pltpu.VMEM((BATCH_BLOCK, PADDED_D), jnp.float32)] * 2,
        [pltpu.VMEM((D_MODEL, BATCH_BLOCK), jnp.float32)] * 2,
        [pltpu.SemaphoreType.DMA] * 2,
        [pltpu.SemaphoreType.DMA] * 2,
        [pltpu.SemaphoreType.DMA] * 2,
    ],
)
def _emb_lookup(ids_hbm, table_hbm, pe_hbm, out_hbm,
                pe_v, idx_l, rows_v, slab_v, sem_i, sem_g, sem_o):
    wid = lax.axis_index("s") * NUM_CORES + lax.axis_index("c")
    bbase = wid * BATCH_BLOCK

    pltpu.sync_copy(pe_hbm, pe_v)

    def issue_idx(l, b):
        pltpu.async_copy(ids_hbm.at[pl.ds(l * B + bbase, BATCH_BLOCK)],
                         idx_l[b], sem_i[b])

    def wait_idx(b):
        pltpu.make_async_copy(ids_hbm.at[pl.ds(0, BATCH_BLOCK)],
                              idx_l[b], sem_i[b]).wait()

    def issue_gather(b):
        pltpu.async_copy(table_hbm.at[idx_l[b]], rows_v[b], sem_g[b])

    def wait_gather(b):
        pltpu.make_async_copy(table_hbm.at[pl.ds(0, BATCH_BLOCK)],
                              rows_v[b], sem_g[b]).wait()

    def wait_out(b):
        pltpu.make_async_copy(slab_v[b],
                              out_hbm.at[0, :, pl.ds(0, BATCH_BLOCK)],
                              sem_o[b]).wait()

    def process(l, b, next_idx, drain_out):
        wait_gather(b)
        if next_idx is not None:
            issue_idx(next_idx, b)
        if drain_out:
            wait_out(b)

        iota = lax.iota(jnp.int32, LANES)
        carry = tuple(pe_v[pl.ds(l * D_MODEL + j * LANES, LANES)]
                      for j in range(JGROUPS)) + tuple(
                          iota + j * LANES for j in range(JGROUPS))

        def tok_body(p, c):
            colv = jnp.zeros((LANES,), jnp.int32) + p
            for j in range(JGROUPS):
                vals = rows_v[b][p, pl.ds(j * LANES, LANES)] * 8.0 + c[j]
                plsc.store_scatter(slab_v[b], [c[JGROUPS + j], colv], vals)
            return c

        lax.fori_loop(0, BATCH_BLOCK, tok_body, carry, unroll=False)
        pltpu.async_copy(slab_v[b],
                         out_hbm.at[l, :, pl.ds(bbase, BATCH_BLOCK)],
                         sem_o[b])

    issue_idx(0, 0)
    issue_idx(1, 1)
    wait_idx(0)
    issue_gather(0)
    wait_idx(1)
    issue_gather(1)
    process(0, 0, next_idx=2, drain_out=False)
    wait_idx(0)
    issue_gather(0)
    process(1, 1, next_idx=3, drain_out=False)

    def pair_body(tt, _):
        l0 = tt * 2
        wait_idx(1)
        issue_gather(1)
        process(l0, 0, next_idx=l0 + 2, drain_out=True)
        wait_idx(0)
        issue_gather(0)
        process(l0 + 1, 1, next_idx=l0 + 3, drain_out=True)
        return _

    lax.fori_loop(1, L // 2 - 1, pair_body, None, unroll=False)

    wait_idx(1)
    issue_gather(1)
    process(L - 2, 0, next_idx=None, drain_out=True)
    process(L - 1, 1, next_idx=None, drain_out=True)
    wait_out(0)
    wait_out(1)


def kernel(input_ids, W):
    ids_lmajor = input_ids.T.reshape(-1).astype(jnp.int32)
    table = jnp.pad(W, ((0, 0), (0, PADDED_D - D_MODEL)))
    out = _emb_lookup(ids_lmajor, table, jnp.asarray(_PE).reshape(-1))
    return jnp.transpose(out, (2, 0, 1))
